# baseline (speedup 1.0000x reference)
import functools
import numpy as np
import jax
import jax.numpy as jnp
from jax import lax
from jax.experimental import pallas as pl
from jax.experimental.pallas import tpu as pltpu
from jax.experimental.pallas import tpu_sc as plsc

B = 16384
NF = 26
PER_FIELD = 100000
D = 16
EPS = 1e-5
INV = np.float32(1.0 / np.sqrt(1.0 + EPS))
OFFSETS = np.arange(NF, dtype=np.int32) * PER_FIELD

NC, NS = 2, 16
NW = NC * NS
CH = 128
NCHUNK = B // CH
CPW = NCHUNK // NW
G16 = CH // 16


def _fm_sc_body(idx_hbm, emb_hbm, lin_hbm, cross_hbm, linsum_hbm,
                idx_v, rows_v, linrows_v, cross_v, lin_v, sem):
    wid = lax.axis_index("s") * NC + lax.axis_index("c")
    for c in range(CPW):
        gc = wid * CPW + c
        row_base = gc * CH
        pltpu.sync_copy(idx_hbm.at[gc], idx_v)
        pltpu.async_copy(emb_hbm.at[idx_v], rows_v, sem).wait()
        pltpu.async_copy(lin_hbm.at[idx_v], linrows_v, sem).wait()

        def row_body(r, carry):
            s = jnp.zeros((16,), jnp.float32)
            sq = jnp.zeros((16,), jnp.float32)
            for f in range(NF):
                e = rows_v[f * CH + r]
                s = s + e
                sq = sq + e * e
            cross_v[r] = 0.5 * (s * s - sq)
            return carry
        lax.fori_loop(0, CH, row_body, 0)

        for g in range(G16):
            acc = jnp.zeros((16,), jnp.float32)
            for f in range(NF):
                acc = acc + linrows_v[pl.ds(f * CH + g * 16, 16)]
            lin_v[pl.ds(g * 16, 16)] = acc

        pltpu.sync_copy(cross_v, cross_hbm.at[pl.ds(row_base, CH)])
        pltpu.sync_copy(lin_v, linsum_hbm.at[pl.ds(row_base, CH)])


_fm_sc = functools.partial(
    pl.kernel,
    out_type=[
        jax.ShapeDtypeStruct((B, D), jnp.float32),
        jax.ShapeDtypeStruct((B,), jnp.float32),
    ],
    mesh=plsc.VectorSubcoreMesh(core_axis_name="c", subcore_axis_name="s"),
    compiler_params=pltpu.CompilerParams(use_tc_tiling_on_sc=False),
    scratch_types=[
        pltpu.VMEM((NF * CH,), jnp.int32),
        pltpu.VMEM((NF * CH, D), jnp.float32),
        pltpu.VMEM((NF * CH,), jnp.float32),
        pltpu.VMEM((CH, D), jnp.float32),
        pltpu.VMEM((CH,), jnp.float32),
        pltpu.SemaphoreType.DMA,
    ],
)(_fm_sc_body)


RB = 2048


def _mlp_body(cross_ref, lin_ref, lb_ref, g0, b0, W1, bb1, g1, bt1,
              W2, bb2, g2, bt2, W3, bb3, out_ref):
    h = cross_ref[...] * (g0[...] * INV) + b0[...]
    z1 = lax.dot_general(h, W1[...], (((1,), (1,)), ((), ())),
                         preferred_element_type=jnp.float32)
    h1 = jnp.maximum((z1 + bb1[...]) * INV * g1[...] + bt1[...], 0.0)
    z2 = lax.dot_general(h1, W2[...], (((1,), (1,)), ((), ())),
                         preferred_element_type=jnp.float32)
    h2 = jnp.maximum((z2 + bb2[...]) * INV * g2[...] + bt2[...], 0.0)
    z3 = jnp.sum(h2 * W3[...], axis=1)
    out_ref[...] = jax.nn.sigmoid(lin_ref[...] + lb_ref[...] + z3 + bb3[...])


def _full(shape):
    return pl.BlockSpec(shape, lambda i: tuple(0 for _ in shape))


_mlp = pl.pallas_call(
    _mlp_body,
    grid=(B // RB,),
    in_specs=[
        pl.BlockSpec((RB, D), lambda i: (i, 0)),
        pl.BlockSpec((RB,), lambda i: (i,)),
        _full((1,)),
        _full((D,)), _full((D,)),
        _full((64, D)), _full((64,)), _full((64,)), _full((64,)),
        _full((32, 64)), _full((32,)), _full((32,)), _full((32,)),
        _full((1, 32)), _full((1,)),
    ],
    out_specs=pl.BlockSpec((RB,), lambda i: (i,)),
    out_shape=jax.ShapeDtypeStruct((B,), jnp.float32),
)


def kernel(users_feat, items_feat, emb_table, lin_table, lin_bias,
           g0, b0, W1, bb1, g1, bt1, W2, bb2, g2, bt2, W3, bb3):
    x = jnp.concatenate([users_feat, items_feat], axis=1) + jnp.asarray(
        OFFSETS, dtype=jnp.int32)
    xc = x.reshape(NCHUNK, CH, NF).transpose(0, 2, 1).reshape(NCHUNK, NF * CH)
    lin1 = lin_table.reshape(-1)
    cross, linsum = _fm_sc(xc, emb_table, lin1)
    return _mlp(cross, linsum, lin_bias, g0, b0, W1, bb1, g1, bt1,
                W2, bb2, g2, bt2, W3, bb3)
